# baseline (speedup 1.0000x reference)
import dataclasses
import functools

import jax
import jax.numpy as jnp
from jax import lax
from jax.experimental import pallas as pl
from jax.experimental.pallas import tpu as pltpu
from jax.experimental.pallas import tpu_sc as plsc

_N = 10000
_E = 320000
_IN = 128
_OUT = 128
_POS = 16
_NPAD = 10240
_NW = 32
_WROWS = _NPAD // _NW
_EPW = _E // _NW
_GR1 = _EPW // 16
_BPW = 672
_BTOT = _NW * _BPW
_BWORDS = _BPW * 16
_NEG = -1e30


def _proj_body(x_ref, pos_ref, w_ref, b_ref, p_ref, q_ref):
    xb = x_ref[...]
    posb = pos_ref[...]
    w1 = w_ref[0:_IN, :]
    w2 = w_ref[_IN:2 * _IN, :]
    w3 = w_ref[2 * _IN:2 * _IN + _POS, :]
    pw = jnp.dot(posb, w3, preferred_element_type=jnp.float32)
    p_ref[...] = (jnp.dot(xb, w1, preferred_element_type=jnp.float32)
                  - pw + b_ref[...])
    q_ref[...] = jnp.dot(xb, w2, preferred_element_type=jnp.float32) + pw


def _tc_proj(x_p, pos_p, w, b2):
    blk = 1024
    return pl.pallas_call(
        _proj_body,
        grid=(_NPAD // blk,),
        in_specs=[
            pl.BlockSpec((blk, _IN), lambda i: (i, 0)),
            pl.BlockSpec((blk, _POS), lambda i: (i, 0)),
            pl.BlockSpec((2 * _IN + _POS, _OUT), lambda i: (0, 0)),
            pl.BlockSpec((1, _OUT), lambda i: (0, 0)),
        ],
        out_specs=[
            pl.BlockSpec((blk, _OUT), lambda i: (i, 0)),
            pl.BlockSpec((blk, _OUT), lambda i: (i, 0)),
        ],
        out_shape=[
            jax.ShapeDtypeStruct((_NPAD, _OUT), jnp.float32),
            jax.ShapeDtypeStruct((_NPAD, _OUT), jnp.float32),
        ],
    )(x_p, pos_p, w, b2)


_CP = pltpu.CompilerParams()
if "needs_layout_passes" in pltpu.CompilerParams.__dataclass_fields__:
    _CP = dataclasses.replace(_CP, needs_layout_passes=False)

_MESH = plsc.VectorSubcoreMesh(core_axis_name="c", subcore_axis_name="s")


def _dyng(x, idx):
    dnums = lax.GatherDimensionNumbers(
        offset_dims=(), collapsed_slice_dims=(0,), start_index_map=(0,))
    return lax.gather(x, idx[:, None], dnums, (1,),
                      mode=lax.GatherScatterMode.PROMISE_IN_BOUNDS)


@functools.partial(
    pl.kernel,
    out_type=[
        jax.ShapeDtypeStruct((_NW * _BWORDS,), jnp.int32),
        jax.ShapeDtypeStruct((_NW * _BWORDS,), jnp.int32),
        jax.ShapeDtypeStruct((_BTOT,), jnp.int32),
        jax.ShapeDtypeStruct((512,), jnp.int32),
    ],
    compiler_params=_CP,
    mesh=_MESH,
    scratch_types=[
        pltpu.VMEM((_EPW,), jnp.int32),
        pltpu.VMEM((_EPW,), jnp.int32),
        pltpu.VMEM((_NW * 32,), jnp.int32),
        pltpu.VMEM((_NW * 32,), jnp.int32),
        pltpu.VMEM((48,), jnp.int32),
        pltpu.VMEM((_BWORDS,), jnp.int32),
        pltpu.VMEM((_BWORDS,), jnp.int32),
        pltpu.VMEM((_BPW,), jnp.int32),
        pltpu.VMEM((16,), jnp.int32),
    ],
)
def _sc_phase1(src_hbm, dst_hbm, bs_hbm, bd_hbm, hdr_hbm, nbl_hbm,
               srcc, dstc, stage_s, stage_d, counts, blk_s, blk_d, hdrv,
               nb16):
    wid = lax.axis_index("s") * 2 + lax.axis_index("c")
    iota16 = lax.iota(jnp.int32, 16)
    ebase = wid * _EPW
    pltpu.sync_copy(src_hbm.at[pl.ds(ebase, _EPW)], srcc)
    pltpu.sync_copy(dst_hbm.at[pl.ds(ebase, _EPW)], dstc)
    counts[pl.ds(0, 16)] = iota16 * 0
    counts[pl.ds(16, 16)] = iota16 * 0
    counts[pl.ds(32, 16)] = iota16 * 0

    def group(g, nblk):
        d16 = dstc[pl.ds(g * 16, 16)]
        s16 = srcc[pl.ds(g * 16, 16)]
        b16 = (d16 * 6554) >> 21
        key = b16 * 16 + iota16
        pack = s16 * 16384 + d16
        kk, vv = plsc.sort_key_val(key, pack)
        bs = kk >> 4
        prev = _dyng(bs, (iota16 - 1) & 15)
        run_start = (bs != prev) | (iota16 == 0)
        start_pos = plsc.cummax(jnp.where(run_start, iota16, 0))
        occ = iota16 - start_pos
        nxt = _dyng(bs, (iota16 + 1) & 15)
        is_last = (bs != nxt) | (iota16 == 15)
        cb = plsc.load_gather(counts, [bs])
        pos = cb + occ
        addr = bs * 32 + (pos & 31)
        plsc.store_scatter(stage_s, [addr], vv >> 14)
        plsc.store_scatter(stage_d, [addr], vv & 16383)
        ca = pos + 1
        il = is_last.astype(jnp.int32)
        caddr = 32 + iota16 + (bs - 32 - iota16) * il
        plsc.store_scatter(counts, [caddr], ca)
        crossed = jnp.where(is_last & ((ca >> 4) > (cb >> 4)), 1, 0)

        def w_body(_, st):
            m, nblk = st
            ffs = plsc.all_reduce_ffs(m != 0)
            bsel = _dyng(bs, ffs)
            casel = _dyng(ca, ffs)
            saddr = bsel * 32 + (((casel >> 4) - 1) & 1) * 16 + iota16
            blk_s[pl.ds(nblk * 16, 16)] = plsc.load_gather(stage_s, [saddr])
            blk_d[pl.ds(nblk * 16, 16)] = plsc.load_gather(stage_d, [saddr])
            plsc.store_scatter(hdrv, [iota16 * 0 + nblk], bsel)
            return m * jnp.where(iota16 == ffs, 0, 1), nblk + 1

        ncross = jnp.sum(crossed)
        _, nblk = lax.fori_loop(0, ncross, w_body, (crossed, nblk))
        return nblk

    nblk = lax.fori_loop(0, _GR1, group, jnp.int32(0))

    def tailb(b, nblk):
        cntv = plsc.load_gather(counts, [iota16 * 0 + b])
        remv = cntv & 15

        def emit(nblk):
            basev = b * 32 + ((cntv >> 4) & 1) * 16 + iota16
            cur_s = plsc.load_gather(stage_s, [basev])
            cur_d = plsc.load_gather(stage_d, [basev])
            pad_s = _dyng(cur_s, remv - 1)
            pad_d = _dyng(cur_d, remv - 1)
            blk_s[pl.ds(nblk * 16, 16)] = jnp.where(iota16 < remv, cur_s,
                                                    pad_s)
            blk_d[pl.ds(nblk * 16, 16)] = jnp.where(iota16 < remv, cur_d,
                                                    pad_d)
            plsc.store_scatter(hdrv, [iota16 * 0 + nblk], iota16 * 0 + b)
            return nblk + 1

        return lax.cond(jnp.max(remv) > 0, emit, lambda n: n, nblk)

    nblk = lax.fori_loop(0, _NW, tailb, nblk)

    pltpu.sync_copy(blk_s, bs_hbm.at[pl.ds(wid * _BWORDS, _BWORDS)])
    pltpu.sync_copy(blk_d, bd_hbm.at[pl.ds(wid * _BWORDS, _BWORDS)])
    pltpu.sync_copy(hdrv, hdr_hbm.at[pl.ds(wid * _BPW, _BPW)])
    nb16[...] = iota16 * 0 + nblk
    pltpu.sync_copy(nb16, nbl_hbm.at[pl.ds(wid * 16, 16)])


@functools.partial(
    pl.kernel,
    out_type=jax.ShapeDtypeStruct((_NPAD * _OUT,), jnp.float32),
    compiler_params=_CP,
    mesh=_MESH,
    scratch_types=[
        pltpu.VMEM((_WROWS * _OUT,), jnp.float32),
        pltpu.VMEM((_WROWS * _OUT,), jnp.float32),
        pltpu.VMEM((_BTOT,), jnp.int32),
        pltpu.VMEM((512,), jnp.int32),
        pltpu.VMEM((16, 128), jnp.int32),
        pltpu.VMEM((16, 128), jnp.int32),
        pltpu.VMEM((16, 128), jnp.int32),
        pltpu.VMEM((16, 128), jnp.int32),
        pltpu.VMEM((128, _OUT), jnp.float32),
        pltpu.SemaphoreType.DMA,
        pltpu.SemaphoreType.DMA,
        pltpu.SemaphoreType.DMA,
        pltpu.SemaphoreType.DMA,
        pltpu.SemaphoreType.DMA,
        pltpu.SemaphoreType.DMA,
        pltpu.SemaphoreType.DMA,
        pltpu.SemaphoreType.DMA,
        pltpu.SemaphoreType.DMA,
        pltpu.SemaphoreType.DMA,
    ],
)
def _sc_phase2(p_hbm, q_hbm, bs2_hbm, bd2_hbm, hdr_hbm, nbl_hbm, out_hbm,
               acc, pbuf, hdrs, nbv, csrc0, cdst0, csrc1, cdst1, rows,
               cb0, cb1, r0, r1, r2, r3, r4, r5, r6, r7):
    wid = lax.axis_index("s") * 2 + lax.axis_index("c")
    lo = wid * _WROWS
    iota16 = lax.iota(jnp.int32, 16)
    rsems = [r0, r1, r2, r3, r4, r5, r6, r7]

    @pl.loop(0, _WROWS * _OUT, step=16)
    def _(j):
        acc[pl.ds(j, 16)] = iota16.astype(jnp.float32) * 0.0 + _NEG

    pltpu.sync_copy(hdr_hbm, hdrs)
    pltpu.sync_copy(nbl_hbm, nbv)

    def fire_rows(srcs, slot):
        def mk(s):
            def f():
                pltpu.async_copy(q_hbm.at[srcs],
                                 rows.at[pl.ds(s * 16, 16)], rsems[s])
            return f

        lax.switch(slot, [mk(i) for i in range(8)])

    def wait_rows(slot):
        def mk(s):
            return lambda: pltpu.make_async_copy(
                q_hbm.at[pl.ds(0, 16)], rows.at[pl.ds(s * 16, 16)],
                rsems[s]).wait()

        lax.switch(slot, [mk(i) for i in range(8)])

    def proc_batch(csrc, cdst, cb, prevpend):
        pltpu.make_async_copy(bs2_hbm.at[pl.ds(0, 16)], csrc, cb).wait()
        pltpu.make_async_copy(bs2_hbm.at[pl.ds(0, 16)], cdst, cb).wait()
        colbase = (prevpend & 7) * 16

        def blk_srcs(bi):
            bsp = iota16 * 0 + bi
            cb16 = _dyng(colbase, bsp) + iota16
            return plsc.load_gather(csrc, [bsp, cb16])

        @pl.loop(0, 8)
        def _(k):
            fire_rows(blk_srcs(k), k)

        @pl.loop(0, 16)
        def _(bi):
            slot = bi & 7
            wait_rows(slot)
            bsp = iota16 * 0 + bi
            cb16 = _dyng(colbase, bsp) + iota16
            dloc = plsc.load_gather(cdst, [bsp, cb16]) - lo

            @pl.loop(0, 16, step=4)
            def _(i0):
                for u in range(4):
                    i = i0 + u
                    dv = _dyng(dloc, iota16 * 0 + i)
                    base = dv * _OUT
                    for c in range(8):
                        a = base + (iota16 + 16 * c)
                        cur = plsc.load_gather(acc, [a])
                        val = rows[slot * 16 + i, pl.ds(16 * c, 16)]
                        plsc.store_scatter(acc, [a], jnp.maximum(cur, val))

            @pl.when(bi < 8)
            def _():
                fire_rows(blk_srcs(bi + 8), slot)

    def flushq(pendfull, par):
        rowidx = pendfull >> 3

        def f0():
            pltpu.async_copy(bs2_hbm.at[rowidx], csrc0, cb0)
            pltpu.async_copy(bd2_hbm.at[rowidx], cdst0, cb0)

        def f1():
            pltpu.async_copy(bs2_hbm.at[rowidx], csrc1, cb1)
            pltpu.async_copy(bd2_hbm.at[rowidx], cdst1, cb1)

        lax.cond(par == 0, f0, f1)

    def proc_prev(prevpend, pvalid, par):
        lax.cond(
            pvalid > 0,
            lambda: lax.cond(
                par == 1,
                lambda: proc_batch(csrc0, cdst0, cb0, prevpend),
                lambda: proc_batch(csrc1, cdst1, cb1, prevpend)),
            lambda: None)

    def scan_w2(w2, carry):
        nbw = jnp.max(nbv[pl.ds(w2 * 16, 16)])
        gmax = (nbw + 15) >> 4
        base = w2 * _BPW

        def g_body(g, carry):
            pend, pcnt, prevp, pvalid, par = carry
            hv = hdrs[pl.ds(base + g * 16, 16)]
            match = (hv == wid) & (g * 16 + iota16 < nbw)
            cnt = jnp.sum(match.astype(jnp.int32))

            def merge(pend, pcnt, prevp, pvalid, par):
                key = jnp.where(match, iota16, iota16 + 16)
                payload = base + g * 16 + iota16
                _, vv = plsc.sort_key_val(key, payload)
                rot = _dyng(vv, (iota16 - pcnt) & 15)
                merged = jnp.where(iota16 < pcnt, pend, rot)
                total = pcnt + cnt

                def full(merged, vv, pcnt, total, prevp, pvalid, par):
                    leftover = _dyng(vv, (iota16 + (16 - pcnt)) & 15)
                    flushq(merged, par)
                    proc_prev(prevp, pvalid, par)
                    return (leftover, total - 16, merged, jnp.int32(1),
                            1 - par)

                def nofull(merged, vv, pcnt, total, prevp, pvalid, par):
                    return merged, total, prevp, pvalid, par

                return lax.cond(total >= 16, full, nofull,
                                merged, vv, pcnt, total, prevp, pvalid, par)

            def skip(pend, pcnt, prevp, pvalid, par):
                return pend, pcnt, prevp, pvalid, par

            return lax.cond(cnt > 0, merge, skip,
                            pend, pcnt, prevp, pvalid, par)

        return lax.fori_loop(0, gmax, g_body, carry)

    carry0 = (iota16 * 0, jnp.int32(0), iota16 * 0, jnp.int32(0),
              jnp.int32(0))
    pend, pcnt, prevp, pvalid, par = lax.fori_loop(0, _NW, scan_w2, carry0)

    proc_prev(prevp, pvalid, par)

    def drain():
        pendp = jnp.where(iota16 < pcnt, pend, _dyng(pend, iota16 * 0))
        flushq(pendp, par)
        proc_prev(pendp, jnp.int32(1), 1 - par)

    lax.cond(pcnt > 0, drain, lambda: None)

    pltpu.sync_copy(p_hbm.at[pl.ds(lo * _OUT, _WROWS * _OUT)], pbuf)

    @pl.loop(0, _WROWS * _OUT, step=16)
    def _(j):
        acc[pl.ds(j, 16)] = jnp.maximum(
            acc[pl.ds(j, 16)] + pbuf[pl.ds(j, 16)], 0.0)

    pltpu.sync_copy(acc, out_hbm.at[pl.ds(lo * _OUT, _WROWS * _OUT)])


def kernel(h, x, pos, edge_index_gate, edge_index_cand, W, b):
    x_p = jnp.pad(x, ((0, _NPAD - _N), (0, 0)))
    pos_p = jnp.pad(pos, ((0, _NPAD - _N), (0, 0)))
    p_mat, q_mat = _tc_proj(x_p, pos_p, W, b.reshape(1, _OUT))
    src = edge_index_cand[0]
    dst = edge_index_cand[1]
    bs_flat, bd_flat, hdr, nbl = _sc_phase1(src, dst)
    out_flat = _sc_phase2(p_mat.reshape(-1), q_mat,
                          bs_flat.reshape(_BTOT // 8, 128),
                          bd_flat.reshape(_BTOT // 8, 128), hdr, nbl)
    return out_flat.reshape(_NPAD, _OUT)[:_N]


# speedup vs baseline: 1.0237x; 1.0237x over previous
import dataclasses
import functools

import jax
import jax.numpy as jnp
from jax import lax
from jax.experimental import pallas as pl
from jax.experimental.pallas import tpu as pltpu
from jax.experimental.pallas import tpu_sc as plsc

_N = 10000
_E = 320000
_IN = 128
_OUT = 128
_POS = 16
_NPAD = 10240
_NW = 32
_WROWS = _NPAD // _NW
_EPW = _E // _NW
_GR1 = _EPW // 16
_BPW = 672
_BTOT = _NW * _BPW
_BWORDS = _BPW * 16
_NEG = -1e30


def _proj_body(x_ref, pos_ref, w_ref, b_ref, p_ref, q_ref):
    xb = x_ref[...]
    posb = pos_ref[...]
    w1 = w_ref[0:_IN, :]
    w2 = w_ref[_IN:2 * _IN, :]
    w3 = w_ref[2 * _IN:2 * _IN + _POS, :]
    pw = jnp.dot(posb, w3, preferred_element_type=jnp.float32)
    p_ref[...] = (jnp.dot(xb, w1, preferred_element_type=jnp.float32)
                  - pw + b_ref[...])
    q_ref[...] = jnp.dot(xb, w2, preferred_element_type=jnp.float32) + pw


def _tc_proj(x_p, pos_p, w, b2):
    blk = 1024
    return pl.pallas_call(
        _proj_body,
        grid=(_NPAD // blk,),
        in_specs=[
            pl.BlockSpec((blk, _IN), lambda i: (i, 0)),
            pl.BlockSpec((blk, _POS), lambda i: (i, 0)),
            pl.BlockSpec((2 * _IN + _POS, _OUT), lambda i: (0, 0)),
            pl.BlockSpec((1, _OUT), lambda i: (0, 0)),
        ],
        out_specs=[
            pl.BlockSpec((blk, _OUT), lambda i: (i, 0)),
            pl.BlockSpec((blk, _OUT), lambda i: (i, 0)),
        ],
        out_shape=[
            jax.ShapeDtypeStruct((_NPAD, _OUT), jnp.float32),
            jax.ShapeDtypeStruct((_NPAD, _OUT), jnp.float32),
        ],
    )(x_p, pos_p, w, b2)


_CP = pltpu.CompilerParams()
if "needs_layout_passes" in pltpu.CompilerParams.__dataclass_fields__:
    _CP = dataclasses.replace(_CP, needs_layout_passes=False)

_MESH = plsc.VectorSubcoreMesh(core_axis_name="c", subcore_axis_name="s")


def _dyng(x, idx):
    dnums = lax.GatherDimensionNumbers(
        offset_dims=(), collapsed_slice_dims=(0,), start_index_map=(0,))
    return lax.gather(x, idx[:, None], dnums, (1,),
                      mode=lax.GatherScatterMode.PROMISE_IN_BOUNDS)


@functools.partial(
    pl.kernel,
    out_type=[
        jax.ShapeDtypeStruct((_NW * _BWORDS,), jnp.int32),
        jax.ShapeDtypeStruct((_NW * _BWORDS,), jnp.int32),
        jax.ShapeDtypeStruct((_BTOT,), jnp.int32),
        jax.ShapeDtypeStruct((512,), jnp.int32),
    ],
    compiler_params=_CP,
    mesh=_MESH,
    scratch_types=[
        pltpu.VMEM((_EPW,), jnp.int32),
        pltpu.VMEM((_EPW,), jnp.int32),
        pltpu.VMEM((_NW * 32,), jnp.int32),
        pltpu.VMEM((_NW * 32,), jnp.int32),
        pltpu.VMEM((48,), jnp.int32),
        pltpu.VMEM((_BWORDS,), jnp.int32),
        pltpu.VMEM((_BWORDS,), jnp.int32),
        pltpu.VMEM((_BPW,), jnp.int32),
        pltpu.VMEM((16,), jnp.int32),
    ],
)
def _sc_phase1(src_hbm, dst_hbm, bs_hbm, bd_hbm, hdr_hbm, nbl_hbm,
               srcc, dstc, stage_s, stage_d, counts, blk_s, blk_d, hdrv,
               nb16):
    wid = lax.axis_index("s") * 2 + lax.axis_index("c")
    iota16 = lax.iota(jnp.int32, 16)
    ebase = wid * _EPW
    pltpu.sync_copy(src_hbm.at[pl.ds(ebase, _EPW)], srcc)
    pltpu.sync_copy(dst_hbm.at[pl.ds(ebase, _EPW)], dstc)
    counts[pl.ds(0, 16)] = iota16 * 0
    counts[pl.ds(16, 16)] = iota16 * 0
    counts[pl.ds(32, 16)] = iota16 * 0

    def group(g, nblk):
        d16 = dstc[pl.ds(g * 16, 16)]
        s16 = srcc[pl.ds(g * 16, 16)]
        b16 = (d16 * 6554) >> 21
        key = b16 * 16 + iota16
        pack = s16 * 16384 + d16
        kk, vv = plsc.sort_key_val(key, pack)
        bs = kk >> 4
        prev = _dyng(bs, (iota16 - 1) & 15)
        run_start = (bs != prev) | (iota16 == 0)
        start_pos = plsc.cummax(jnp.where(run_start, iota16, 0))
        occ = iota16 - start_pos
        nxt = _dyng(bs, (iota16 + 1) & 15)
        is_last = (bs != nxt) | (iota16 == 15)
        cb = plsc.load_gather(counts, [bs])
        pos = cb + occ
        addr = bs * 32 + (pos & 31)
        plsc.store_scatter(stage_s, [addr], vv >> 14)
        plsc.store_scatter(stage_d, [addr], vv & 16383)
        ca = pos + 1
        il = is_last.astype(jnp.int32)
        caddr = 32 + iota16 + (bs - 32 - iota16) * il
        plsc.store_scatter(counts, [caddr], ca)
        crossed = jnp.where(is_last & ((ca >> 4) > (cb >> 4)), 1, 0)

        def w_body(_, st):
            m, nblk = st
            ffs = plsc.all_reduce_ffs(m != 0)
            bsel = _dyng(bs, ffs)
            casel = _dyng(ca, ffs)
            saddr = bsel * 32 + (((casel >> 4) - 1) & 1) * 16 + iota16
            blk_s[pl.ds(nblk * 16, 16)] = plsc.load_gather(stage_s, [saddr])
            blk_d[pl.ds(nblk * 16, 16)] = plsc.load_gather(stage_d, [saddr])
            plsc.store_scatter(hdrv, [iota16 * 0 + nblk], bsel)
            return m * jnp.where(iota16 == ffs, 0, 1), nblk + 1

        ncross = jnp.sum(crossed)
        _, nblk = lax.fori_loop(0, ncross, w_body, (crossed, nblk))
        return nblk

    nblk = lax.fori_loop(0, _GR1, group, jnp.int32(0))

    def tailb(b, nblk):
        cntv = plsc.load_gather(counts, [iota16 * 0 + b])
        remv = cntv & 15

        def emit(nblk):
            basev = b * 32 + ((cntv >> 4) & 1) * 16 + iota16
            cur_s = plsc.load_gather(stage_s, [basev])
            cur_d = plsc.load_gather(stage_d, [basev])
            pad_s = _dyng(cur_s, remv - 1)
            pad_d = _dyng(cur_d, remv - 1)
            blk_s[pl.ds(nblk * 16, 16)] = jnp.where(iota16 < remv, cur_s,
                                                    pad_s)
            blk_d[pl.ds(nblk * 16, 16)] = jnp.where(iota16 < remv, cur_d,
                                                    pad_d)
            plsc.store_scatter(hdrv, [iota16 * 0 + nblk], iota16 * 0 + b)
            return nblk + 1

        return lax.cond(jnp.max(remv) > 0, emit, lambda n: n, nblk)

    nblk = lax.fori_loop(0, _NW, tailb, nblk)

    pltpu.sync_copy(blk_s, bs_hbm.at[pl.ds(wid * _BWORDS, _BWORDS)])
    pltpu.sync_copy(blk_d, bd_hbm.at[pl.ds(wid * _BWORDS, _BWORDS)])
    pltpu.sync_copy(hdrv, hdr_hbm.at[pl.ds(wid * _BPW, _BPW)])
    nb16[...] = iota16 * 0 + nblk
    pltpu.sync_copy(nb16, nbl_hbm.at[pl.ds(wid * 16, 16)])


@functools.partial(
    pl.kernel,
    out_type=jax.ShapeDtypeStruct((_NPAD * _OUT,), jnp.float32),
    compiler_params=_CP,
    mesh=_MESH,
    scratch_types=[
        pltpu.VMEM((_WROWS * _OUT,), jnp.float32),
        pltpu.VMEM((_WROWS * _OUT,), jnp.float32),
        pltpu.VMEM((_BTOT,), jnp.int32),
        pltpu.VMEM((512,), jnp.int32),
        pltpu.VMEM((16, 128), jnp.int32),
        pltpu.VMEM((16, 128), jnp.int32),
        pltpu.VMEM((16, 128), jnp.int32),
        pltpu.VMEM((16, 128), jnp.int32),
        pltpu.VMEM((64, _OUT), jnp.float32),
        pltpu.SemaphoreType.DMA,
        pltpu.SemaphoreType.DMA,
        pltpu.SemaphoreType.DMA,
        pltpu.SemaphoreType.DMA,
        pltpu.SemaphoreType.DMA,
        pltpu.SemaphoreType.DMA,
    ],
)
def _sc_phase2(p_hbm, q_hbm, bs2_hbm, bd2_hbm, hdr_hbm, nbl_hbm, out_hbm,
               acc, pbuf, hdrs, nbv, csrc0, cdst0, csrc1, cdst1, rows,
               cb0, cb1, r0, r1, r2, r3):
    wid = lax.axis_index("s") * 2 + lax.axis_index("c")
    lo = wid * _WROWS
    iota16 = lax.iota(jnp.int32, 16)
    rsems = [r0, r1, r2, r3]

    @pl.loop(0, _WROWS * _OUT, step=16)
    def _(j):
        acc[pl.ds(j, 16)] = iota16.astype(jnp.float32) * 0.0 + _NEG

    pltpu.sync_copy(hdr_hbm, hdrs)
    pltpu.sync_copy(nbl_hbm, nbv)

    def fire_rows(srcs, slot):
        def mk(s):
            def f():
                pltpu.async_copy(q_hbm.at[srcs],
                                 rows.at[pl.ds(s * 16, 16)], rsems[s])
            return f

        lax.switch(slot, [mk(0), mk(1), mk(2), mk(3)])

    def wait_rows(slot):
        def mk(s):
            return lambda: pltpu.make_async_copy(
                q_hbm.at[pl.ds(0, 16)], rows.at[pl.ds(s * 16, 16)],
                rsems[s]).wait()

        lax.switch(slot, [mk(0), mk(1), mk(2), mk(3)])

    def proc_batch(csrc, cdst, cb, prevpend):
        pltpu.make_async_copy(bs2_hbm.at[pl.ds(0, 16)], csrc, cb).wait()
        pltpu.make_async_copy(bs2_hbm.at[pl.ds(0, 16)], cdst, cb).wait()
        colbase = (prevpend & 7) * 16

        def blk_srcs(bi):
            bsp = iota16 * 0 + bi
            cb16 = _dyng(colbase, bsp) + iota16
            return plsc.load_gather(csrc, [bsp, cb16])

        @pl.loop(0, 4)
        def _(k):
            fire_rows(blk_srcs(k), k)

        @pl.loop(0, 16)
        def _(bi):
            slot = bi & 3
            wait_rows(slot)
            bsp = iota16 * 0 + bi
            cb16 = _dyng(colbase, bsp) + iota16
            dloc = plsc.load_gather(cdst, [bsp, cb16]) - lo

            @pl.loop(0, 16, step=4)
            def _(i0):
                for u in range(4):
                    i = i0 + u
                    dv = _dyng(dloc, iota16 * 0 + i)
                    base = dv * _OUT
                    for c in range(8):
                        a = base + (iota16 + 16 * c)
                        cur = plsc.load_gather(acc, [a])
                        val = rows[slot * 16 + i, pl.ds(16 * c, 16)]
                        plsc.store_scatter(acc, [a], jnp.maximum(cur, val))

            @pl.when(bi < 12)
            def _():
                fire_rows(blk_srcs(bi + 4), slot)

    def flushq(pendfull, par):
        rowidx = pendfull >> 3

        def f0():
            pltpu.async_copy(bs2_hbm.at[rowidx], csrc0, cb0)
            pltpu.async_copy(bd2_hbm.at[rowidx], cdst0, cb0)

        def f1():
            pltpu.async_copy(bs2_hbm.at[rowidx], csrc1, cb1)
            pltpu.async_copy(bd2_hbm.at[rowidx], cdst1, cb1)

        lax.cond(par == 0, f0, f1)

    def proc_prev(prevpend, pvalid, par):
        lax.cond(
            pvalid > 0,
            lambda: lax.cond(
                par == 1,
                lambda: proc_batch(csrc0, cdst0, cb0, prevpend),
                lambda: proc_batch(csrc1, cdst1, cb1, prevpend)),
            lambda: None)

    def scan_w2(w2, carry):
        nbw = jnp.max(nbv[pl.ds(w2 * 16, 16)])
        gmax = (nbw + 15) >> 4
        base = w2 * _BPW

        def g_body(g, carry):
            pend, pcnt, prevp, pvalid, par = carry
            hv = hdrs[pl.ds(base + g * 16, 16)]
            match = (hv == wid) & (g * 16 + iota16 < nbw)
            cnt = jnp.sum(match.astype(jnp.int32))

            def merge(pend, pcnt, prevp, pvalid, par):
                key = jnp.where(match, iota16, iota16 + 16)
                payload = base + g * 16 + iota16
                _, vv = plsc.sort_key_val(key, payload)
                rot = _dyng(vv, (iota16 - pcnt) & 15)
                merged = jnp.where(iota16 < pcnt, pend, rot)
                total = pcnt + cnt

                def full(merged, vv, pcnt, total, prevp, pvalid, par):
                    leftover = _dyng(vv, (iota16 + (16 - pcnt)) & 15)
                    flushq(merged, par)
                    proc_prev(prevp, pvalid, par)
                    return (leftover, total - 16, merged, jnp.int32(1),
                            1 - par)

                def nofull(merged, vv, pcnt, total, prevp, pvalid, par):
                    return merged, total, prevp, pvalid, par

                return lax.cond(total >= 16, full, nofull,
                                merged, vv, pcnt, total, prevp, pvalid, par)

            def skip(pend, pcnt, prevp, pvalid, par):
                return pend, pcnt, prevp, pvalid, par

            return lax.cond(cnt > 0, merge, skip,
                            pend, pcnt, prevp, pvalid, par)

        return lax.fori_loop(0, gmax, g_body, carry)

    carry0 = (iota16 * 0, jnp.int32(0), iota16 * 0, jnp.int32(0),
              jnp.int32(0))
    pend, pcnt, prevp, pvalid, par = lax.fori_loop(0, _NW, scan_w2, carry0)

    proc_prev(prevp, pvalid, par)

    def drain():
        pendp = jnp.where(iota16 < pcnt, pend, _dyng(pend, iota16 * 0))
        flushq(pendp, par)
        proc_prev(pendp, jnp.int32(1), 1 - par)

    lax.cond(pcnt > 0, drain, lambda: None)

    pltpu.sync_copy(p_hbm.at[pl.ds(lo * _OUT, _WROWS * _OUT)], pbuf)

    @pl.loop(0, _WROWS * _OUT, step=16)
    def _(j):
        acc[pl.ds(j, 16)] = jnp.maximum(
            acc[pl.ds(j, 16)] + pbuf[pl.ds(j, 16)], 0.0)

    pltpu.sync_copy(acc, out_hbm.at[pl.ds(lo * _OUT, _WROWS * _OUT)])


def kernel(h, x, pos, edge_index_gate, edge_index_cand, W, b):
    x_p = jnp.pad(x, ((0, _NPAD - _N), (0, 0)))
    pos_p = jnp.pad(pos, ((0, _NPAD - _N), (0, 0)))
    p_mat, q_mat = _tc_proj(x_p, pos_p, W, b.reshape(1, _OUT))
    src = edge_index_cand[0]
    dst = edge_index_cand[1]
    bs_flat, bd_flat, hdr, nbl = _sc_phase1(src, dst)
    out_flat = _sc_phase2(p_mat.reshape(-1), q_mat,
                          bs_flat.reshape(_BTOT // 8, 128),
                          bd_flat.reshape(_BTOT // 8, 128), hdr, nbl)
    return out_flat.reshape(_NPAD, _OUT)[:_N]
